# Initial kernel scaffold; baseline (speedup 1.0000x reference)
import jax
import jax.numpy as jnp
from jax.experimental import pallas as pl


def kernel(cdd_repr, his_repr, his_embedding, his_attn_mask, his_mask, W, b):
    raise NotImplementedError("write your pallas kernel here")


# speedup vs baseline: 4.8812x; 4.8812x over previous
import functools

import jax
import jax.numpy as jnp
from jax import lax
from jax.experimental import pallas as pl
from jax.experimental.pallas import tpu as pltpu
from jax.experimental.pallas import tpu_sc as plsc

B = 1024
CDD = 5
HIS = 50
SL = 32
ED = 32
HD = 128
K = 5

NEG_INF = float("-inf")


def _proj_norm_body(his_ref, cdd_ref, wt_ref, b_ref, hnt_ref, cn_ref):
    w = wt_ref[...]
    bv = b_ref[0:1, :]
    yh = jnp.dot(his_ref[...], w) + bv
    yt = yh.T
    s = jnp.sum(yt * yt, axis=0, keepdims=True)
    hnt_ref[...] = yt / jnp.maximum(jnp.sqrt(s), 1e-12)
    yc = jnp.dot(cdd_ref[...], w) + bv
    yct = yc.T
    sc = jnp.sum(yct * yct, axis=0, keepdims=True)
    cn_ref[...] = (yct / jnp.maximum(jnp.sqrt(sc), 1e-12)).T


def _proj_norm(his2, cdd2, wt, b8, *, interpret=False):
    grid = 8
    hr = his2.shape[0] // grid
    cr = cdd2.shape[0] // grid
    return pl.pallas_call(
        _proj_norm_body,
        grid=(grid,),
        in_specs=[
            pl.BlockSpec((hr, HD), lambda i: (i, 0)),
            pl.BlockSpec((cr, HD), lambda i: (i, 0)),
            pl.BlockSpec((HD, HD), lambda i: (0, 0)),
            pl.BlockSpec((8, HD), lambda i: (0, 0)),
        ],
        out_specs=[
            pl.BlockSpec((HD, hr), lambda i: (0, i)),
            pl.BlockSpec((cr, HD), lambda i: (i, 0)),
        ],
        out_shape=[
            jax.ShapeDtypeStruct((HD, his2.shape[0]), jnp.float32),
            jax.ShapeDtypeStruct((cdd2.shape[0], HD), jnp.float32),
        ],
        interpret=interpret,
    )(his2, cdd2, wt, b8)


_G = 8
_M = 16
_BLK = _G * _M
_RB = _BLK * CDD


def _topk_body(cn_ref, hnt_ref, hm_ref, idx_ref):
    parts = []
    for m in range(_M):
        lhs = cn_ref[m * _G * CDD:(m + 1) * _G * CDD, :]
        rhs = hnt_ref[:, m * _G * HIS:(m + 1) * _G * HIS]
        full = jnp.dot(lhs, rhs)
        riota = lax.broadcasted_iota(jnp.int32, (_G * CDD, 1), 0) // CDD
        acc = jnp.zeros((_G * CDD, HIS), jnp.float32)
        for g in range(_G):
            acc = acc + full[:, g * HIS:(g + 1) * HIS] * (
                riota == g).astype(jnp.float32)
        parts.append(acc)
    a = jnp.concatenate(parts, axis=0)
    hm = hm_ref[...]
    iot = lax.broadcasted_iota(jnp.int32, (_RB, HIS), 1)
    keep = (iot < K).astype(jnp.float32)
    padpos = (hm + keep) == 0.0
    a = jnp.where(padpos, NEG_INF, a)
    bnum = (pl.program_id(0) * _RB
            + lax.broadcasted_iota(jnp.int32, (_RB, 1), 0)) // CDD
    rowbase = bnum * HIS
    cols = []
    for _ in range(K):
        mx = jnp.max(a, axis=-1, keepdims=True)
        sel = jnp.min(jnp.where(a == mx, iot, HIS), axis=-1, keepdims=True)
        cols.append(rowbase + sel)
        a = jnp.where(iot == sel, NEG_INF, a)
    idx_ref[...] = jnp.concatenate(cols, axis=1)


def _attn_topk(cn2, hnt, hm_bc, *, interpret=False):
    grid = B // _BLK
    return pl.pallas_call(
        _topk_body,
        grid=(grid,),
        in_specs=[
            pl.BlockSpec((_RB, HD), lambda i: (i, 0)),
            pl.BlockSpec((HD, _BLK * HIS), lambda i: (0, i)),
            pl.BlockSpec((_RB, HIS), lambda i: (i, 0)),
        ],
        out_specs=pl.BlockSpec((_RB, K), lambda i: (i, 0)),
        out_shape=jax.ShapeDtypeStruct((B * CDD, K), jnp.int32),
        interpret=interpret,
    )(cn2, hnt, hm_bc)


def _mask_sel_body(hamt_ref, idx_ref, out_ref, *, bb):
    hamt = hamt_ref[...]
    iot = lax.broadcasted_iota(jnp.int32, (bb, HIS), 1)
    rowbase = (pl.program_id(0) * bb
               + lax.broadcasted_iota(jnp.int32, (bb, 1), 0)) * HIS
    mcols = []
    for ck in range(CDD * K):
        sel = idx_ref[:, ck:ck + 1] - rowbase
        onehot = (iot == sel).astype(jnp.float32)
        mcols.append(jnp.sum(hamt * onehot[:, None, :], axis=-1))
    out_ref[...] = jnp.concatenate(mcols, axis=1)


def _mask_sel(hamt, fidx2, *, interpret=False):
    bb = 256
    grid = B // bb
    return pl.pallas_call(
        functools.partial(_mask_sel_body, bb=bb),
        grid=(grid,),
        in_specs=[
            pl.BlockSpec((bb, SL, HIS), lambda i: (i, 0, 0)),
            pl.BlockSpec((bb, CDD * K), lambda i: (i, 0)),
        ],
        out_specs=pl.BlockSpec((bb, CDD * K * SL), lambda i: (i, 0)),
        out_shape=jax.ShapeDtypeStruct((B, CDD * K * SL), jnp.float32),
        interpret=interpret,
    )(hamt, fidx2)


_NC = 2
_NS = 16
_NW = _NC * _NS
_ROWS = B * CDD * K
_RPW = _ROWS // _NW
_CH = 32
_NCH = _RPW // _CH


def _sc_gather_body(emb_hbm, idx_hbm, out_e_hbm, idx_v, ebuf, esem):
    wid = lax.axis_index("s") * _NC + lax.axis_index("c")
    pltpu.sync_copy(idx_hbm.at[wid], idx_v)
    for g in range(_NCH):
        pltpu.async_copy(emb_hbm.at[idx_v.at[g]], ebuf, esem).wait()
        pltpu.sync_copy(ebuf, out_e_hbm.at[wid].at[g])


def _sc_gather(emb_tab, idx3):
    mesh = plsc.VectorSubcoreMesh(core_axis_name="c", subcore_axis_name="s")
    fn = pl.kernel(
        _sc_gather_body,
        out_type=jax.ShapeDtypeStruct((_NW, _NCH, _CH, SL * ED), jnp.float32),
        mesh=mesh,
        scratch_types=[
            pltpu.VMEM((_NCH, _CH), jnp.int32),
            pltpu.VMEM((_CH, SL * ED), jnp.float32),
            pltpu.SemaphoreType.DMA,
        ],
    )
    return fn(emb_tab, idx3)


def kernel(cdd_repr, his_repr, his_embedding, his_attn_mask, his_mask, W, b):
    his2 = his_repr.reshape(B * HIS, HD)
    cdd2 = cdd_repr.reshape(B * CDD, HD)
    wt = W.T
    b8 = jnp.broadcast_to(b.reshape(1, HD), (8, HD))

    hnt, cn2 = _proj_norm(his2, cdd2, wt, b8)

    hm_bc = jnp.repeat(his_mask.reshape(B, HIS), CDD, axis=0)
    fidx = _attn_topk(cn2, hnt, hm_bc)

    hamt = jnp.swapaxes(his_attn_mask, 1, 2)
    mout = _mask_sel(hamt, fidx.reshape(B, CDD * K))

    emb_tab = his_embedding.reshape(B * HIS, SL * ED)
    idx3 = fidx.reshape(_NW, _NCH, _CH)
    out_e = _sc_gather(emb_tab, idx3)

    his_selected = out_e.reshape(B, CDD, K, SL, ED)
    his_mask_selected = mout.reshape(B, CDD, K, SL)
    return (his_selected, his_mask_selected)


# speedup vs baseline: 8.2523x; 1.6906x over previous
import functools

import jax
import jax.numpy as jnp
from jax import lax
from jax.experimental import pallas as pl
from jax.experimental.pallas import tpu as pltpu
from jax.experimental.pallas import tpu_sc as plsc

B = 1024
CDD = 5
HIS = 50
SL = 32
ED = 32
HD = 128
K = 5

NEG_INF = float("-inf")


def _proj_norm_body(his_ref, cdd_ref, wt_ref, b_ref, hnt_ref, cn_ref):
    w = wt_ref[...]
    bv = b_ref[0:1, :]
    yh = jnp.dot(his_ref[...], w) + bv
    yt = yh.T
    s = jnp.sum(yt * yt, axis=0, keepdims=True)
    hnt_ref[...] = yt / jnp.maximum(jnp.sqrt(s), 1e-12)
    yc = jnp.dot(cdd_ref[...], w) + bv
    yct = yc.T
    sc = jnp.sum(yct * yct, axis=0, keepdims=True)
    cn_ref[...] = (yct / jnp.maximum(jnp.sqrt(sc), 1e-12)).T


def _proj_norm(his2, cdd2, wt, b8, *, interpret=False):
    grid = 8
    hr = his2.shape[0] // grid
    cr = cdd2.shape[0] // grid
    return pl.pallas_call(
        _proj_norm_body,
        grid=(grid,),
        in_specs=[
            pl.BlockSpec((hr, HD), lambda i: (i, 0)),
            pl.BlockSpec((cr, HD), lambda i: (i, 0)),
            pl.BlockSpec((HD, HD), lambda i: (0, 0)),
            pl.BlockSpec((8, HD), lambda i: (0, 0)),
        ],
        out_specs=[
            pl.BlockSpec((HD, hr), lambda i: (0, i)),
            pl.BlockSpec((cr, HD), lambda i: (i, 0)),
        ],
        out_shape=[
            jax.ShapeDtypeStruct((HD, his2.shape[0]), jnp.float32),
            jax.ShapeDtypeStruct((cdd2.shape[0], HD), jnp.float32),
        ],
        interpret=interpret,
    )(his2, cdd2, wt, b8)


_G = 8
_M = 16
_BLK = _G * _M
_RB = _BLK * CDD


def _topk_body(cn_ref, hnt_ref, hm_ref, idx_ref):
    parts = []
    for m in range(_M):
        lhs = cn_ref[m * _G * CDD:(m + 1) * _G * CDD, :]
        rhs = hnt_ref[:, m * _G * HIS:(m + 1) * _G * HIS]
        full = jnp.dot(lhs, rhs)
        riota = lax.broadcasted_iota(jnp.int32, (_G * CDD, 1), 0) // CDD
        acc = jnp.zeros((_G * CDD, HIS), jnp.float32)
        for g in range(_G):
            acc = acc + full[:, g * HIS:(g + 1) * HIS] * (
                riota == g).astype(jnp.float32)
        parts.append(acc)
    a = jnp.concatenate(parts, axis=0)
    hm = hm_ref[...]
    iot = lax.broadcasted_iota(jnp.int32, (_RB, HIS), 1)
    keep = (iot < K).astype(jnp.float32)
    padpos = (hm + keep) == 0.0
    a = jnp.where(padpos, NEG_INF, a)
    cols = []
    for _ in range(K):
        mx = jnp.max(a, axis=-1, keepdims=True)
        sel = jnp.min(jnp.where(a == mx, iot, HIS), axis=-1, keepdims=True)
        cols.append(sel)
        a = jnp.where(iot == sel, NEG_INF, a)
    idx_ref[...] = jnp.concatenate(cols, axis=1)


def _attn_topk(cn2, hnt, hm_bc, *, interpret=False):
    grid = B // _BLK
    return pl.pallas_call(
        _topk_body,
        grid=(grid,),
        in_specs=[
            pl.BlockSpec((_RB, HD), lambda i: (i, 0)),
            pl.BlockSpec((HD, _BLK * HIS), lambda i: (0, i)),
            pl.BlockSpec((_RB, HIS), lambda i: (i, 0)),
        ],
        out_specs=pl.BlockSpec((_RB, K), lambda i: (i, 0)),
        out_shape=jax.ShapeDtypeStruct((B * CDD, K), jnp.int32),
        interpret=interpret,
    )(cn2, hnt, hm_bc)


_NC = 2
_NS = 16
_NW = _NC * _NS
_CK = CDD * K
_HB = 512
_SEW = (SL * ED) // _NW


def _gather_halves(tab_ref, out_ref, idx_v, slab, obuf, dsem, se_lo, se_n):
    zvec = jnp.zeros((16,), jnp.int32)
    for half in range(2):
        b0 = half * _HB

        def se_body(i, _):
            se = se_lo + i
            pltpu.async_copy(
                tab_ref.at[:, pl.ds(se, 1), pl.ds(b0, _HB)], slab, dsem
            ).wait()

            def ck_body(ck, _):
                for v in range(_HB // 16):
                    hvec = idx_v[ck, pl.ds(b0 + v * 16, 16)]
                    cvec = lax.broadcasted_iota(jnp.int32, (16,), 0) + v * 16
                    vals = plsc.load_gather(slab, [hvec, zvec, cvec])
                    obuf[ck, 0, pl.ds(v * 16, 16)] = vals
                return 0

            lax.fori_loop(0, _CK, ck_body, 0)
            pltpu.sync_copy(
                obuf, out_ref.at[:, pl.ds(se, 1), pl.ds(b0, _HB)]
            )
            return 0

        lax.fori_loop(0, se_n, se_body, 0)


def _sc_gather_body(emb_hbm, msk_hbm, idxt_hbm, out_e_hbm, out_m_hbm,
                    idx_v, slab, obuf, dsem):
    wid = lax.axis_index("s") * _NC + lax.axis_index("c")
    pltpu.sync_copy(idxt_hbm, idx_v)
    _gather_halves(emb_hbm, out_e_hbm, idx_v, slab, obuf, dsem,
                   wid * _SEW, _SEW)
    _gather_halves(msk_hbm, out_m_hbm, idx_v, slab, obuf, dsem, wid, 1)


def _sc_gather(emb3, msk3, idxt):
    mesh = plsc.VectorSubcoreMesh(core_axis_name="c", subcore_axis_name="s")
    fn = pl.kernel(
        _sc_gather_body,
        out_type=[
            jax.ShapeDtypeStruct((_CK, SL * ED, B), jnp.float32),
            jax.ShapeDtypeStruct((_CK, SL, B), jnp.float32),
        ],
        mesh=mesh,
        scratch_types=[
            pltpu.VMEM((_CK, B), jnp.int32),
            pltpu.VMEM((HIS, 1, _HB), jnp.float32),
            pltpu.VMEM((_CK, 1, _HB), jnp.float32),
            pltpu.SemaphoreType.DMA,
        ],
        compiler_params=pltpu.CompilerParams(needs_layout_passes=False),
    )
    return fn(emb3, msk3, idxt)


def kernel(cdd_repr, his_repr, his_embedding, his_attn_mask, his_mask, W, b):
    his2 = his_repr.reshape(B * HIS, HD)
    cdd2 = cdd_repr.reshape(B * CDD, HD)
    wt = W.T
    b8 = jnp.broadcast_to(b.reshape(1, HD), (8, HD))

    hnt, cn2 = _proj_norm(his2, cdd2, wt, b8)

    hm_bc = jnp.repeat(his_mask.reshape(B, HIS), CDD, axis=0)
    lidx = _attn_topk(cn2, hnt, hm_bc)

    idxt = lidx.reshape(B, _CK).T

    emb3 = jnp.transpose(his_embedding, (1, 2, 3, 0)).reshape(HIS, SL * ED, B)
    msk3 = jnp.transpose(his_attn_mask, (1, 2, 0))

    out_e, out_m = _sc_gather(emb3, msk3, idxt)

    his_selected = jnp.transpose(
        out_e.reshape(CDD, K, SL, ED, B), (4, 0, 1, 2, 3))
    his_mask_selected = jnp.transpose(
        out_m.reshape(CDD, K, SL, B), (3, 0, 1, 2))
    return (his_selected, his_mask_selected)


# speedup vs baseline: 9.7958x; 1.1870x over previous
import functools

import jax
import jax.numpy as jnp
from jax import lax
from jax.experimental import pallas as pl
from jax.experimental.pallas import tpu as pltpu
from jax.experimental.pallas import tpu_sc as plsc

B = 1024
CDD = 5
HIS = 50
SL = 32
ED = 32
HD = 128
K = 5

NEG_INF = float("-inf")


def _proj_norm_body(his_ref, cdd_ref, wt_ref, b_ref, hnt_ref, cn_ref):
    w = wt_ref[...]
    bv = b_ref[0:1, :]
    yh = jnp.dot(his_ref[...], w) + bv
    yt = yh.T
    s = jnp.sum(yt * yt, axis=0, keepdims=True)
    hnt_ref[...] = yt / jnp.maximum(jnp.sqrt(s), 1e-12)
    yc = jnp.dot(cdd_ref[...], w) + bv
    yct = yc.T
    sc = jnp.sum(yct * yct, axis=0, keepdims=True)
    cn_ref[...] = (yct / jnp.maximum(jnp.sqrt(sc), 1e-12)).T


def _proj_norm(his2, cdd2, wt, b8, *, interpret=False):
    grid = 8
    hr = his2.shape[0] // grid
    cr = cdd2.shape[0] // grid
    return pl.pallas_call(
        _proj_norm_body,
        grid=(grid,),
        in_specs=[
            pl.BlockSpec((hr, HD), lambda i: (i, 0)),
            pl.BlockSpec((cr, HD), lambda i: (i, 0)),
            pl.BlockSpec((HD, HD), lambda i: (0, 0)),
            pl.BlockSpec((8, HD), lambda i: (0, 0)),
        ],
        out_specs=[
            pl.BlockSpec((HD, hr), lambda i: (0, i)),
            pl.BlockSpec((cr, HD), lambda i: (i, 0)),
        ],
        out_shape=[
            jax.ShapeDtypeStruct((HD, his2.shape[0]), jnp.float32),
            jax.ShapeDtypeStruct((cdd2.shape[0], HD), jnp.float32),
        ],
        interpret=interpret,
    )(his2, cdd2, wt, b8)


_G = 8
_M = 16
_BLK = _G * _M
_RB = _BLK * CDD


def _topk_body(cn_ref, hnt_ref, hm_ref, idx_ref):
    parts = []
    for m in range(_M):
        lhs = cn_ref[m * _G * CDD:(m + 1) * _G * CDD, :]
        rhs = hnt_ref[:, m * _G * HIS:(m + 1) * _G * HIS]
        full = jnp.dot(lhs, rhs)
        riota = lax.broadcasted_iota(jnp.int32, (_G * CDD, 1), 0) // CDD
        acc = jnp.zeros((_G * CDD, HIS), jnp.float32)
        for g in range(_G):
            acc = acc + full[:, g * HIS:(g + 1) * HIS] * (
                riota == g).astype(jnp.float32)
        parts.append(acc)
    a = jnp.concatenate(parts, axis=0)
    hm = hm_ref[...]
    iot = lax.broadcasted_iota(jnp.int32, (_RB, HIS), 1)
    keep = (iot < K).astype(jnp.float32)
    padpos = (hm + keep) == 0.0
    a = jnp.where(padpos, NEG_INF, a)
    cols = []
    for _ in range(K):
        mx = jnp.max(a, axis=-1, keepdims=True)
        sel = jnp.min(jnp.where(a == mx, iot, HIS), axis=-1, keepdims=True)
        cols.append(sel)
        a = jnp.where(iot == sel, NEG_INF, a)
    idx_ref[...] = jnp.concatenate(cols, axis=1)


def _attn_topk(cn2, hnt, hm_bc, *, interpret=False):
    grid = B // _BLK
    return pl.pallas_call(
        _topk_body,
        grid=(grid,),
        in_specs=[
            pl.BlockSpec((_RB, HD), lambda i: (i, 0)),
            pl.BlockSpec((HD, _BLK * HIS), lambda i: (0, i)),
            pl.BlockSpec((_RB, HIS), lambda i: (i, 0)),
        ],
        out_specs=pl.BlockSpec((_RB, K), lambda i: (i, 0)),
        out_shape=jax.ShapeDtypeStruct((B * CDD, K), jnp.int32),
        interpret=interpret,
    )(cn2, hnt, hm_bc)


_NC = 2
_NS = 16
_NW = _NC * _NS
_CK = CDD * K
_HB = 512
_SEW = (SL * ED) // _NW


def _gather_halves(tab_ref, out_ref, idx_v, slab, obuf, isem, osem,
                   se_lo, se_n):
    zvec = jnp.zeros((16,), jnp.int32)
    cvecs = [lax.broadcasted_iota(jnp.int32, (16,), 0) + v * 16
             for v in range(_HB // 16)]
    for half in range(2):
        b0 = half * _HB

        def start_in(i, par):
            pltpu.async_copy(
                tab_ref.at[:, pl.ds(se_lo + i, 1), pl.ds(b0, _HB)],
                slab.at[par], isem)

        start_in(0, 0)

        def se_body(i, _):
            par = lax.rem(i, 2)
            pltpu.make_async_copy(
                tab_ref.at[:, pl.ds(se_lo, 1), pl.ds(b0, _HB)],
                slab.at[par], isem).wait()

            @pl.when(i + 1 < se_n)
            def _():
                start_in(i + 1, 1 - par)

            @pl.when(i >= 2)
            def _():
                pltpu.make_async_copy(
                    obuf.at[par],
                    out_ref.at[:, pl.ds(se_lo, 1), pl.ds(b0, _HB)],
                    osem).wait()

            sl = slab.at[par]
            ob = obuf.at[par]

            def ck_body(ck, _):
                for v in range(_HB // 16):
                    hvec = idx_v[ck, pl.ds(b0 + v * 16, 16)]
                    vals = plsc.load_gather(sl, [hvec, zvec, cvecs[v]])
                    ob[ck, 0, pl.ds(v * 16, 16)] = vals
                return 0

            lax.fori_loop(0, _CK, ck_body, 0)
            pltpu.async_copy(
                obuf.at[par],
                out_ref.at[:, pl.ds(se_lo + i, 1), pl.ds(b0, _HB)], osem)
            return 0

        lax.fori_loop(0, se_n, se_body, 0)
        for j in range(2 if se_n >= 2 else 1):
            pltpu.make_async_copy(
                obuf.at[j],
                out_ref.at[:, pl.ds(se_lo, 1), pl.ds(b0, _HB)], osem).wait()


def _sc_gather_body(emb_hbm, msk_hbm, idxt_hbm, out_e_hbm, out_m_hbm,
                    idx_v, slab, obuf, isem, osem):
    wid = lax.axis_index("s") * _NC + lax.axis_index("c")
    pltpu.sync_copy(idxt_hbm, idx_v)
    _gather_halves(emb_hbm, out_e_hbm, idx_v, slab, obuf, isem, osem,
                   wid * _SEW, _SEW)
    _gather_halves(msk_hbm, out_m_hbm, idx_v, slab, obuf, isem, osem, wid, 1)


def _sc_gather(emb3, msk3, idxt):
    mesh = plsc.VectorSubcoreMesh(core_axis_name="c", subcore_axis_name="s")
    fn = pl.kernel(
        _sc_gather_body,
        out_type=[
            jax.ShapeDtypeStruct((_CK, SL * ED, B), jnp.float32),
            jax.ShapeDtypeStruct((_CK, SL, B), jnp.float32),
        ],
        mesh=mesh,
        scratch_types=[
            pltpu.VMEM((_CK, B), jnp.int32),
            pltpu.VMEM((2, HIS, 1, _HB), jnp.float32),
            pltpu.VMEM((2, _CK, 1, _HB), jnp.float32),
            pltpu.SemaphoreType.DMA,
            pltpu.SemaphoreType.DMA,
        ],
        compiler_params=pltpu.CompilerParams(needs_layout_passes=False),
    )
    return fn(emb3, msk3, idxt)


def kernel(cdd_repr, his_repr, his_embedding, his_attn_mask, his_mask, W, b):
    his2 = his_repr.reshape(B * HIS, HD)
    cdd2 = cdd_repr.reshape(B * CDD, HD)
    wt = W.T
    b8 = jnp.broadcast_to(b.reshape(1, HD), (8, HD))

    hnt, cn2 = _proj_norm(his2, cdd2, wt, b8)

    hm_bc = jnp.repeat(his_mask.reshape(B, HIS), CDD, axis=0)
    lidx = _attn_topk(cn2, hnt, hm_bc)

    idxt = lidx.reshape(B, _CK).T

    emb3 = jnp.transpose(his_embedding, (1, 2, 3, 0)).reshape(HIS, SL * ED, B)
    msk3 = jnp.transpose(his_attn_mask, (1, 2, 0))

    out_e, out_m = _sc_gather(emb3, msk3, idxt)

    his_selected = jnp.transpose(
        out_e.reshape(CDD, K, SL, ED, B), (4, 0, 1, 2, 3))
    his_mask_selected = jnp.transpose(
        out_m.reshape(CDD, K, SL, B), (3, 0, 1, 2))
    return (his_selected, his_mask_selected)
